# baseline (speedup 1.0000x reference)
import jax
import jax.numpy as jnp
from jax import lax
from jax.experimental import pallas as pl
from jax.experimental.pallas import tpu as pltpu

ROWS = 16384
COLS = 1000
BLK_R = 256
GRID = ROWS // BLK_R
LW = 128


def _body(lab_ref, x_ref, out_ref):
    lab_col = lab_ref[:, 0:1]
    col = lax.broadcasted_iota(jnp.int32, (BLK_R, COLS), 1)
    v = jnp.sum(
        jnp.where(col == lab_col, x_ref[...], 0.0), axis=1, keepdims=True
    )
    res = jnp.exp(jnp.abs(v) * 0.1) - 1.0
    out_ref[...] = lax.broadcast_in_dim(res, (BLK_R, LW), (0, 1))


@jax.jit
def kernel(x, labels):
    lab128 = jnp.broadcast_to(
        labels.astype(jnp.int32)[:, None], (ROWS, LW)
    )
    out128 = pl.pallas_call(
        _body,
        grid=(GRID,),
        in_specs=[
            pl.BlockSpec((BLK_R, LW), lambda g: (g, 0)),
            pl.BlockSpec((BLK_R, COLS), lambda g: (g, 0)),
        ],
        out_specs=pl.BlockSpec((BLK_R, LW), lambda g: (g, 0)),
        out_shape=jax.ShapeDtypeStruct((ROWS, LW), jnp.float32),
        compiler_params=pltpu.CompilerParams(
            dimension_semantics=("arbitrary",)
        ),
    )(lab128, x)
    return out128[:, :1]


# speedup vs baseline: 1.2660x; 1.2660x over previous
import jax
import jax.numpy as jnp
from jax import lax
from jax.experimental import pallas as pl
from jax.experimental.pallas import tpu as pltpu

ROWS = 16384
COLS = 1000
BLK_R = 1024
GRID = ROWS // BLK_R
LW = 128


def _body(lab_ref, x_ref, out_ref):
    lab_col = lab_ref[:, 0:1]
    col = lax.broadcasted_iota(jnp.int32, (BLK_R, COLS), 1)
    v = jnp.sum(
        jnp.where(col == lab_col, x_ref[...], 0.0), axis=1, keepdims=True
    )
    res = jnp.exp(jnp.abs(v) * 0.1) - 1.0
    out_ref[...] = lax.broadcast_in_dim(res, (BLK_R, LW), (0, 1))


@jax.jit
def kernel(x, labels):
    lab128 = jnp.broadcast_to(
        labels.astype(jnp.int32)[:, None], (ROWS, LW)
    )
    out128 = pl.pallas_call(
        _body,
        grid=(GRID,),
        in_specs=[
            pl.BlockSpec((BLK_R, LW), lambda g: (g, 0)),
            pl.BlockSpec((BLK_R, COLS), lambda g: (g, 0)),
        ],
        out_specs=pl.BlockSpec((BLK_R, LW), lambda g: (g, 0)),
        out_shape=jax.ShapeDtypeStruct((ROWS, LW), jnp.float32),
        compiler_params=pltpu.CompilerParams(
            dimension_semantics=("parallel",)
        ),
    )(lab128, x)
    return out128[:, :1]
